# Initial kernel scaffold; baseline (speedup 1.0000x reference)
import jax
import jax.numpy as jnp
from jax.experimental import pallas as pl


def kernel(atom_feats, coords_t, bond_index, bond_feats, num_atoms, Wq, Wk, Wv, WeK, WeV, Wo, W1, W2, g1, b1, g2, b2):
    raise NotImplementedError("write your pallas kernel here")


# speedup vs baseline: 11.2271x; 11.2271x over previous
import functools
import math

import jax
import jax.numpy as jnp
from jax import lax
from jax.experimental import pallas as pl
from jax.experimental.pallas import tpu as pltpu

NM = 200
A = 50
N = NM * A
DM = 256
H = 8
DH = DM // H
G = 64
EB = 20000
MAXNEI = 24
MAXR = 5.0
R2 = MAXR * MAXR
CH = 512
EPAD = EB + CH
DFF = 1024
RS = 1.0 / math.sqrt(DH)
NEG = -1e30


def _mol_kernel(starts_ref, h_ref, c_ref, sl_ref, tl_ref, bf_ref,
                wq_ref, wk_ref, wv_ref, wekb_ref, wekd_ref,
                wevb_ref, wevd_ref, out_ref):
    i = pl.program_id(0)
    X = h_ref[0]
    c = c_ref[0]

    d2 = jnp.zeros((A, A), jnp.float32)
    for j in range(3):
        cj = c[:, j:j + 1]
        dj = cj - jnp.transpose(cj)
        d2 = d2 + dj * dj
    d = jnp.sqrt(d2 + 1e-12)

    iota = lax.broadcasted_iota(jnp.int32, (A, A), 1)
    iota_r = lax.broadcasted_iota(jnp.int32, (A, A), 0)
    work = d2
    xsel = jnp.zeros((A, A), jnp.bool_)
    for _ in range(MAXNEI):
        mval = jnp.min(work, axis=1, keepdims=True)
        ismin = work == mval
        fidx = jnp.min(jnp.where(ismin, iota, A), axis=1, keepdims=True)
        first = iota == fidx
        xsel = jnp.logical_or(xsel, first)
        work = jnp.where(first, 1e30, work)
    within = d2 <= R2
    xw = jnp.logical_and(xsel, within)
    nout = jnp.sum(jnp.where(jnp.logical_and(xsel, jnp.logical_not(within)),
                             1.0, 0.0), axis=1, keepdims=True)
    diag = iota == iota_r
    M = xw.astype(jnp.float32) + jnp.where(diag, nout, 0.0)
    valid = M > 0.0

    step = MAXR / (G - 1)
    coeff = -0.5 / (step * step)
    offs = lax.broadcasted_iota(jnp.int32, (A, A, G), 2).astype(jnp.float32) * step
    dd = d[:, :, None] - offs
    E3 = jnp.exp(coeff * dd * dd)

    Q = jnp.dot(X, wq_ref[...], preferred_element_type=jnp.float32)
    K = jnp.dot(X, wk_ref[...], preferred_element_type=jnp.float32)
    V = jnp.dot(X, wv_ref[...], preferred_element_type=jnp.float32)

    c11 = (((1,), (1,)), ((), ()))
    c00 = (((0,), (0,)), ((), ()))

    Ls = []
    QPBs = []
    for h in range(H):
        sl_ = slice(h * DH, (h + 1) * DH)
        Qh = Q[:, sl_]
        Kh = K[:, sl_]
        qk = lax.dot_general(Qh, Kh, c11, preferred_element_type=jnp.float32)
        qpk = lax.dot_general(Qh, wekd_ref[:, sl_], c11,
                              preferred_element_type=jnp.float32)
        l2 = jnp.sum(E3 * qpk[:, None, :], axis=2)
        Ls.append((qk + l2) * RS)
        QPBs.append(lax.dot_general(Qh, wekb_ref[:, sl_], c11,
                                    preferred_element_type=jnp.float32))

    mrad = jnp.concatenate(
        [jnp.transpose(jnp.max(jnp.where(valid, Ls[h], NEG),
                               axis=1, keepdims=True)) for h in range(H)],
        axis=0)

    start = starts_ref[i]
    end = starts_ref[i + 1]
    nch = (end - start + CH - 1) // CH
    iota_c = lax.broadcasted_iota(jnp.int32, (CH, 1), 0)
    iota_ca = lax.broadcasted_iota(jnp.int32, (CH, A), 1)

    def _chunk(ci):
        off = start + ci * CH
        slc = sl_ref[pl.ds(off, CH), :]
        tlc = tl_ref[pl.ds(off, CH), :]
        bfc = bf_ref[pl.ds(off, CH), :]
        vmask = (iota_c + ci * CH) < (end - start)
        oneT = jnp.logical_and(tlc == iota_ca, vmask)
        oneTf = oneT.astype(jnp.float32)
        oneSf = (slc == iota_ca).astype(jnp.float32)
        return oneT, oneTf, oneSf, bfc, vmask

    def _lb(h, oneTf, oneSf, bfc):
        tmp = jnp.dot(oneTf, Ls[h], preferred_element_type=jnp.float32)
        lg = jnp.sum(tmp * oneSf, axis=1, keepdims=True)
        gq = jnp.dot(oneTf, QPBs[h], preferred_element_type=jnp.float32)
        delta = jnp.sum(gq * bfc, axis=1, keepdims=True)
        return lg + delta * RS

    def body1(ci, mcarry):
        oneT, oneTf, oneSf, bfc, _ = _chunk(ci)
        rows = []
        for h in range(H):
            lb = _lb(h, oneTf, oneSf, bfc)
            masked = jnp.where(oneT, lb, NEG)
            rows.append(jnp.maximum(mcarry[h:h + 1, :],
                                    jnp.max(masked, axis=0, keepdims=True)))
        return jnp.concatenate(rows, axis=0)

    m_all = lax.fori_loop(0, nch, body1, mrad)
    mT = jnp.transpose(m_all)

    def body2(ci, carry):
        pc, gc = carry
        _, oneTf, oneSf, bfc, vmask = _chunk(ci)
        mg = jnp.dot(oneTf, mT, preferred_element_type=jnp.float32)
        pparts = []
        gparts = []
        for h in range(H):
            lb = _lb(h, oneTf, oneSf, bfc)
            nh = jnp.where(vmask, jnp.exp(lb - mg[:, h:h + 1]), 0.0)
            pparts.append(lax.dot_general(oneTf, oneSf * nh, c00,
                                          preferred_element_type=jnp.float32))
            gparts.append(lax.dot_general(oneTf, bfc * nh, c00,
                                          preferred_element_type=jnp.float32))
        return (pc + jnp.concatenate(pparts, axis=0),
                gc + jnp.concatenate(gparts, axis=1))

    p0 = jnp.zeros((H * A, A), jnp.float32)
    g0 = jnp.zeros((A, 4 * H), jnp.float32)
    pcat, gcat = lax.fori_loop(0, nch, body2, (p0, g0))

    outs = []
    for h in range(H):
        sl_ = slice(h * DH, (h + 1) * DH)
        mcol = jnp.transpose(m_all[h:h + 1, :])
        expo = jnp.where(valid, Ls[h] - mcol, NEG)
        ph = M * jnp.exp(expo) + pcat[h * A:(h + 1) * A, :]
        ssum = jnp.sum(ph, axis=1, keepdims=True)
        msg = jnp.dot(ph, V[:, sl_], preferred_element_type=jnp.float32)
        th = jnp.sum(ph[:, :, None] * E3, axis=1)
        msg = msg + jnp.dot(th, wevd_ref[:, sl_],
                            preferred_element_type=jnp.float32)
        msg = msg + jnp.dot(gcat[:, h * 4:(h + 1) * 4], wevb_ref[:, sl_],
                            preferred_element_type=jnp.float32)
        outs.append(msg / (ssum + 1e-9))
    out_ref[0] = jnp.concatenate(outs, axis=1)


def _post_kernel(h_ref, msg_ref, wo_ref, w1_ref, w2_ref,
                 g1_ref, b1_ref, g2_ref, b2_ref, out_ref):
    x = h_ref[...]
    h2 = x + jnp.dot(msg_ref[...], wo_ref[...],
                     preferred_element_type=jnp.float32)
    mu = jnp.mean(h2, axis=1, keepdims=True)
    xc = h2 - mu
    var = jnp.mean(xc * xc, axis=1, keepdims=True)
    h2 = xc / jnp.sqrt(var + 1e-5) * g1_ref[...] + b1_ref[...]
    ff = jnp.dot(jnp.maximum(
        jnp.dot(h2, w1_ref[...], preferred_element_type=jnp.float32), 0.0),
        w2_ref[...], preferred_element_type=jnp.float32)
    y = h2 + ff
    mu = jnp.mean(y, axis=1, keepdims=True)
    yc = y - mu
    var = jnp.mean(yc * yc, axis=1, keepdims=True)
    out_ref[...] = yc / jnp.sqrt(var + 1e-5) * g2_ref[...] + b2_ref[...]


@jax.jit
def _run(atom_feats, coords_t, bond_index, bond_feats,
         Wq, Wk, Wv, WeK, WeV, Wo, W1, W2, g1, b1, g2, b2):
    src = bond_index[0]
    tgt = bond_index[1]
    mol = tgt // A
    order = jnp.argsort(mol)
    sl = (src[order] % A).astype(jnp.int32)
    tl = (tgt[order] % A).astype(jnp.int32)
    bf = bond_feats[order]
    starts = jnp.searchsorted(mol[order], jnp.arange(NM + 1)
                              ).astype(jnp.int32)
    pad = EPAD - EB
    sl2 = jnp.pad(sl, (0, pad)).reshape(EPAD, 1)
    tl2 = jnp.pad(tl, (0, pad)).reshape(EPAD, 1)
    bf2 = jnp.pad(bf, ((0, pad), (0, 0)))

    hmol = atom_feats.reshape(NM, A, DM)
    cmol = coords_t.reshape(NM, A, 3)
    wekb, wekd = WeK[:4], WeK[4:]
    wevb, wevd = WeV[:4], WeV[4:]

    const = lambda i, s: (0, 0)
    mol_spec = pltpu.PrefetchScalarGridSpec(
        num_scalar_prefetch=1,
        grid=(NM,),
        in_specs=[
            pl.BlockSpec((1, A, DM), lambda i, s: (i, 0, 0)),
            pl.BlockSpec((1, A, 3), lambda i, s: (i, 0, 0)),
            pl.BlockSpec((EPAD, 1), const),
            pl.BlockSpec((EPAD, 1), const),
            pl.BlockSpec((EPAD, 4), const),
            pl.BlockSpec((DM, DM), const),
            pl.BlockSpec((DM, DM), const),
            pl.BlockSpec((DM, DM), const),
            pl.BlockSpec((4, DM), const),
            pl.BlockSpec((G, DM), const),
            pl.BlockSpec((4, DM), const),
            pl.BlockSpec((G, DM), const),
        ],
        out_specs=pl.BlockSpec((1, A, DM), lambda i, s: (i, 0, 0)),
    )
    msg = pl.pallas_call(
        _mol_kernel,
        grid_spec=mol_spec,
        out_shape=jax.ShapeDtypeStruct((NM, A, DM), jnp.float32),
    )(starts, hmol, cmol, sl2, tl2, bf2, Wq, Wk, Wv,
      wekb, wekd, wevb, wevd).reshape(N, DM)

    RB = 1000
    cst = lambda i: (0, 0)
    out = pl.pallas_call(
        _post_kernel,
        grid=(N // RB,),
        in_specs=[
            pl.BlockSpec((RB, DM), lambda i: (i, 0)),
            pl.BlockSpec((RB, DM), lambda i: (i, 0)),
            pl.BlockSpec((DM, DM), cst),
            pl.BlockSpec((DM, DFF), cst),
            pl.BlockSpec((DFF, DM), cst),
            pl.BlockSpec((1, DM), cst),
            pl.BlockSpec((1, DM), cst),
            pl.BlockSpec((1, DM), cst),
            pl.BlockSpec((1, DM), cst),
        ],
        out_specs=pl.BlockSpec((RB, DM), lambda i: (i, 0)),
        out_shape=jax.ShapeDtypeStruct((N, DM), jnp.float32),
    )(atom_feats, msg, Wo, W1, W2,
      g1.reshape(1, DM), b1.reshape(1, DM),
      g2.reshape(1, DM), b2.reshape(1, DM))
    return out


def kernel(atom_feats, coords_t, bond_index, bond_feats, num_atoms,
           Wq, Wk, Wv, WeK, WeV, Wo, W1, W2, g1, b1, g2, b2):
    return _run(atom_feats, coords_t, bond_index.astype(jnp.int32),
                bond_feats, Wq, Wk, Wv, WeK, WeV, Wo, W1, W2,
                g1, b1, g2, b2)


# speedup vs baseline: 11.2341x; 1.0006x over previous
import functools
import math

import jax
import jax.numpy as jnp
from jax import lax
from jax.experimental import pallas as pl
from jax.experimental.pallas import tpu as pltpu

NM = 200
A = 50
N = NM * A
DM = 256
H = 8
DH = DM // H
G = 64
EB = 20000
MAXNEI = 24
MAXR = 5.0
R2 = MAXR * MAXR
CH = 512
EPAD = EB + CH
DFF = 1024
RS = 1.0 / math.sqrt(DH)
NEG = -1e30


def _mol_kernel(starts_ref, h_ref, c_ref, sl_ref, tl_ref, bf_ref,
                wq_ref, wk_ref, wv_ref, wekb_ref, wekd_ref,
                wevb_ref, wevd_ref, out_ref):
    i = pl.program_id(0)
    X = h_ref[0]
    c = c_ref[0]

    d2 = jnp.zeros((A, A), jnp.float32)
    for j in range(3):
        cj = c[:, j:j + 1]
        dj = cj - jnp.transpose(cj)
        d2 = d2 + dj * dj
    d = jnp.sqrt(d2 + 1e-12)

    iota = lax.broadcasted_iota(jnp.int32, (A, A), 1)
    iota_r = lax.broadcasted_iota(jnp.int32, (A, A), 0)
    work = d2
    xsel = jnp.zeros((A, A), jnp.bool_)
    for _ in range(MAXNEI):
        mval = jnp.min(work, axis=1, keepdims=True)
        ismin = work == mval
        fidx = jnp.min(jnp.where(ismin, iota, A), axis=1, keepdims=True)
        first = iota == fidx
        xsel = jnp.logical_or(xsel, first)
        work = jnp.where(first, 1e30, work)
    within = d2 <= R2
    xw = jnp.logical_and(xsel, within)
    nout = jnp.sum(jnp.where(jnp.logical_and(xsel, jnp.logical_not(within)),
                             1.0, 0.0), axis=1, keepdims=True)
    diag = iota == iota_r
    M = xw.astype(jnp.float32) + jnp.where(diag, nout, 0.0)
    valid = M > 0.0

    step = MAXR / (G - 1)
    coeff = -0.5 / (step * step)
    offs = lax.broadcasted_iota(jnp.int32, (A, A, G), 2).astype(jnp.float32) * step
    dd = d[:, :, None] - offs
    E3 = jnp.exp(coeff * dd * dd)

    Q = jnp.dot(X, wq_ref[...], preferred_element_type=jnp.float32)
    K = jnp.dot(X, wk_ref[...], preferred_element_type=jnp.float32)
    V = jnp.dot(X, wv_ref[...], preferred_element_type=jnp.float32)

    c11 = (((1,), (1,)), ((), ()))
    c00 = (((0,), (0,)), ((), ()))

    Ls = []
    QPBs = []
    for h in range(H):
        sl_ = slice(h * DH, (h + 1) * DH)
        Qh = Q[:, sl_]
        Kh = K[:, sl_]
        qk = lax.dot_general(Qh, Kh, c11, preferred_element_type=jnp.float32)
        qpk = lax.dot_general(Qh, wekd_ref[:, sl_], c11,
                              preferred_element_type=jnp.float32)
        l2 = jnp.sum(E3 * qpk[:, None, :], axis=2)
        Ls.append((qk + l2) * RS)
        QPBs.append(lax.dot_general(Qh, wekb_ref[:, sl_], c11,
                                    preferred_element_type=jnp.float32))

    mrad = jnp.concatenate(
        [jnp.transpose(jnp.max(jnp.where(valid, Ls[h], NEG),
                               axis=1, keepdims=True)) for h in range(H)],
        axis=0)

    start = starts_ref[i]
    end = starts_ref[i + 1]
    nch = (end - start + CH - 1) // CH
    iota_c = lax.broadcasted_iota(jnp.int32, (CH, 1), 0)
    iota_ca = lax.broadcasted_iota(jnp.int32, (CH, A), 1)

    def _chunk(ci):
        off = start + ci * CH
        slc = sl_ref[pl.ds(off, CH), :]
        tlc = tl_ref[pl.ds(off, CH), :]
        bfc = bf_ref[pl.ds(off, CH), :]
        vmask = (iota_c + ci * CH) < (end - start)
        oneT = jnp.logical_and(tlc == iota_ca, vmask)
        oneTf = oneT.astype(jnp.float32)
        oneSf = (slc == iota_ca).astype(jnp.float32)
        return oneT, oneTf, oneSf, bfc, vmask

    def _lb(h, oneTf, oneSf, bfc):
        tmp = jnp.dot(oneTf, Ls[h], preferred_element_type=jnp.float32)
        lg = jnp.sum(tmp * oneSf, axis=1, keepdims=True)
        gq = jnp.dot(oneTf, QPBs[h], preferred_element_type=jnp.float32)
        delta = jnp.sum(gq * bfc, axis=1, keepdims=True)
        return lg + delta * RS

    def body1(ci, mcarry):
        oneT, oneTf, oneSf, bfc, _ = _chunk(ci)
        rows = []
        for h in range(H):
            lb = _lb(h, oneTf, oneSf, bfc)
            masked = jnp.where(oneT, lb, NEG)
            rows.append(jnp.maximum(mcarry[h:h + 1, :],
                                    jnp.max(masked, axis=0, keepdims=True)))
        return jnp.concatenate(rows, axis=0)

    m_all = lax.fori_loop(0, nch, body1, mrad)
    mT = jnp.transpose(m_all)

    def body2(ci, carry):
        pc, gc = carry
        _, oneTf, oneSf, bfc, vmask = _chunk(ci)
        mg = jnp.dot(oneTf, mT, preferred_element_type=jnp.float32)
        pparts = []
        gparts = []
        for h in range(H):
            lb = _lb(h, oneTf, oneSf, bfc)
            nh = jnp.where(vmask, jnp.exp(lb - mg[:, h:h + 1]), 0.0)
            pparts.append(lax.dot_general(oneTf, oneSf * nh, c00,
                                          preferred_element_type=jnp.float32))
            gparts.append(lax.dot_general(oneTf, bfc * nh, c00,
                                          preferred_element_type=jnp.float32))
        return (pc + jnp.concatenate(pparts, axis=0),
                gc + jnp.concatenate(gparts, axis=1))

    p0 = jnp.zeros((H * A, A), jnp.float32)
    g0 = jnp.zeros((A, 4 * H), jnp.float32)
    pcat, gcat = lax.fori_loop(0, nch, body2, (p0, g0))

    outs = []
    for h in range(H):
        sl_ = slice(h * DH, (h + 1) * DH)
        mcol = jnp.transpose(m_all[h:h + 1, :])
        expo = jnp.where(valid, Ls[h] - mcol, NEG)
        ph = M * jnp.exp(expo) + pcat[h * A:(h + 1) * A, :]
        ssum = jnp.sum(ph, axis=1, keepdims=True)
        msg = jnp.dot(ph, V[:, sl_], preferred_element_type=jnp.float32)
        th = jnp.sum(ph[:, :, None] * E3, axis=1)
        msg = msg + jnp.dot(th, wevd_ref[:, sl_],
                            preferred_element_type=jnp.float32)
        msg = msg + jnp.dot(gcat[:, h * 4:(h + 1) * 4], wevb_ref[:, sl_],
                            preferred_element_type=jnp.float32)
        outs.append(msg / (ssum + 1e-9))
    out_ref[0] = jnp.concatenate(outs, axis=1)


def _post_kernel(h_ref, msg_ref, wo_ref, w1_ref, w2_ref,
                 g1_ref, b1_ref, g2_ref, b2_ref, out_ref):
    x = h_ref[...]
    h2 = x + jnp.dot(msg_ref[...], wo_ref[...],
                     preferred_element_type=jnp.float32)
    mu = jnp.mean(h2, axis=1, keepdims=True)
    xc = h2 - mu
    var = jnp.mean(xc * xc, axis=1, keepdims=True)
    h2 = xc / jnp.sqrt(var + 1e-5) * g1_ref[...] + b1_ref[...]
    ff = jnp.dot(jnp.maximum(
        jnp.dot(h2, w1_ref[...], preferred_element_type=jnp.float32), 0.0),
        w2_ref[...], preferred_element_type=jnp.float32)
    y = h2 + ff
    mu = jnp.mean(y, axis=1, keepdims=True)
    yc = y - mu
    var = jnp.mean(yc * yc, axis=1, keepdims=True)
    out_ref[...] = yc / jnp.sqrt(var + 1e-5) * g2_ref[...] + b2_ref[...]


@jax.jit
def _run(atom_feats, coords_t, bond_index, bond_feats,
         Wq, Wk, Wv, WeK, WeV, Wo, W1, W2, g1, b1, g2, b2):
    src = bond_index[0]
    tgt = bond_index[1]
    mol = tgt // A
    order = jnp.argsort(mol)
    sl = (src[order] % A).astype(jnp.int32)
    tl = (tgt[order] % A).astype(jnp.int32)
    bf = bond_feats[order]
    starts = jnp.searchsorted(mol[order], jnp.arange(NM + 1)
                              ).astype(jnp.int32)
    pad = EPAD - EB
    sl2 = jnp.pad(sl, (0, pad)).reshape(EPAD, 1)
    tl2 = jnp.pad(tl, (0, pad)).reshape(EPAD, 1)
    bf2 = jnp.pad(bf, ((0, pad), (0, 0)))

    hmol = atom_feats.reshape(NM, A, DM)
    cmol = coords_t.reshape(NM, A, 3)
    wekb, wekd = WeK[:4], WeK[4:]
    wevb, wevd = WeV[:4], WeV[4:]

    const = lambda i, s: (0, 0)
    mol_spec = pltpu.PrefetchScalarGridSpec(
        num_scalar_prefetch=1,
        grid=(NM,),
        in_specs=[
            pl.BlockSpec((1, A, DM), lambda i, s: (i, 0, 0)),
            pl.BlockSpec((1, A, 3), lambda i, s: (i, 0, 0)),
            pl.BlockSpec((EPAD, 1), const),
            pl.BlockSpec((EPAD, 1), const),
            pl.BlockSpec((EPAD, 4), const),
            pl.BlockSpec((DM, DM), const),
            pl.BlockSpec((DM, DM), const),
            pl.BlockSpec((DM, DM), const),
            pl.BlockSpec((4, DM), const),
            pl.BlockSpec((G, DM), const),
            pl.BlockSpec((4, DM), const),
            pl.BlockSpec((G, DM), const),
        ],
        out_specs=pl.BlockSpec((1, A, DM), lambda i, s: (i, 0, 0)),
    )
    msg = pl.pallas_call(
        _mol_kernel,
        grid_spec=mol_spec,
        out_shape=jax.ShapeDtypeStruct((NM, A, DM), jnp.float32),
        compiler_params=pltpu.CompilerParams(
            dimension_semantics=("parallel",)),
    )(starts, hmol, cmol, sl2, tl2, bf2, Wq, Wk, Wv,
      wekb, wekd, wevb, wevd).reshape(N, DM)

    RB = 1000
    cst = lambda i: (0, 0)
    out = pl.pallas_call(
        _post_kernel,
        grid=(N // RB,),
        in_specs=[
            pl.BlockSpec((RB, DM), lambda i: (i, 0)),
            pl.BlockSpec((RB, DM), lambda i: (i, 0)),
            pl.BlockSpec((DM, DM), cst),
            pl.BlockSpec((DM, DFF), cst),
            pl.BlockSpec((DFF, DM), cst),
            pl.BlockSpec((1, DM), cst),
            pl.BlockSpec((1, DM), cst),
            pl.BlockSpec((1, DM), cst),
            pl.BlockSpec((1, DM), cst),
        ],
        out_specs=pl.BlockSpec((RB, DM), lambda i: (i, 0)),
        out_shape=jax.ShapeDtypeStruct((N, DM), jnp.float32),
        compiler_params=pltpu.CompilerParams(
            dimension_semantics=("parallel",)),
    )(atom_feats, msg, Wo, W1, W2,
      g1.reshape(1, DM), b1.reshape(1, DM),
      g2.reshape(1, DM), b2.reshape(1, DM))
    return out


def kernel(atom_feats, coords_t, bond_index, bond_feats, num_atoms,
           Wq, Wk, Wv, WeK, WeV, Wo, W1, W2, g1, b1, g2, b2):
    return _run(atom_feats, coords_t, bond_index.astype(jnp.int32),
                bond_feats, Wq, Wk, Wv, WeK, WeV, Wo, W1, W2,
                g1, b1, g2, b2)
